# baseline (speedup 1.0000x reference)
import jax
import jax.numpy as jnp
from jax.experimental import pallas as pl
from jax.experimental.pallas import tpu as pltpu

EPS = 1e-12
N_SKILLS = 8
RANK = 16
TS = 1024


def _router_body(task_ids_ref, ml_ref, w_ref):
    bsz = w_ref.shape[0]
    for b in range(bsz):
        tid = task_ids_ref[b]
        row = ml_ref[pl.ds(tid, 1), :]
        p = jax.nn.sigmoid(row)
        w_ref[pl.ds(b, 1), :] = p / (jnp.sum(p) + EPS)


def _fused_body(wts_ref, x_ref, w_ref, bias_ref, la_ref, lb_ref, out_ref,
                weff_ref):
    b = pl.program_id(0)
    s = pl.program_id(1)

    @pl.when(s == 0)
    def _build_weff():
        A = la_ref[0] * wts_ref[b, 0]
        Bm = lb_ref[0] * wts_ref[b, 0]
        for k in range(1, N_SKILLS):
            A = A + la_ref[k] * wts_ref[b, k]
            Bm = Bm + lb_ref[k] * wts_ref[b, k]
        abT = jax.lax.dot_general(
            Bm, A, (((0,), (1,)), ((), ())),
            preferred_element_type=jnp.float32)
        weff_ref[...] = w_ref[...] + abT * (1.0 / RANK)

    out_ref[0] = jax.lax.dot_general(
        x_ref[0], weff_ref[...], (((1,), (1,)), ((), ())),
        preferred_element_type=jnp.float32) + bias_ref[...]


def kernel(x, task_ids, module_logits, weight, bias, lora_a, lora_b):
    bsz, seq, in_f = x.shape
    out_f = weight.shape[0]
    n_tasks, n_sk = module_logits.shape

    wts = pl.pallas_call(
        _router_body,
        in_specs=[
            pl.BlockSpec(memory_space=pltpu.SMEM),
            pl.BlockSpec(memory_space=pltpu.VMEM),
        ],
        out_specs=pl.BlockSpec(memory_space=pltpu.VMEM),
        out_shape=jax.ShapeDtypeStruct((bsz, n_sk), jnp.float32),
    )(task_ids.astype(jnp.int32), module_logits)

    bias2 = bias.reshape(1, out_f)
    la = lora_a.reshape(n_sk, in_f, RANK)
    lb = lora_b.reshape(n_sk, RANK, out_f)

    n_s = seq // TS
    out = pl.pallas_call(
        _fused_body,
        grid=(bsz, n_s),
        in_specs=[
            pl.BlockSpec(memory_space=pltpu.SMEM),
            pl.BlockSpec((1, TS, in_f), lambda b, s: (b, s, 0)),
            pl.BlockSpec((out_f, in_f), lambda b, s: (0, 0)),
            pl.BlockSpec((1, out_f), lambda b, s: (0, 0)),
            pl.BlockSpec((n_sk, in_f, RANK), lambda b, s: (0, 0, 0)),
            pl.BlockSpec((n_sk, RANK, out_f), lambda b, s: (0, 0, 0)),
        ],
        out_specs=pl.BlockSpec((1, TS, out_f), lambda b, s: (b, s, 0)),
        out_shape=jax.ShapeDtypeStruct((bsz, seq, out_f), jnp.float32),
        scratch_shapes=[pltpu.VMEM((out_f, in_f), jnp.float32)],
    )(wts, x, weight, bias2, la, lb)
    return out


# speedup vs baseline: 1.0530x; 1.0530x over previous
import jax
import jax.numpy as jnp
from jax.experimental import pallas as pl
from jax.experimental.pallas import tpu as pltpu

EPS = 1e-12
N_SKILLS = 8
RANK = 16
TS = 2048


def _router_body(task_ids_ref, ml_ref, w_ref):
    bsz = w_ref.shape[0]
    for b in range(bsz):
        tid = task_ids_ref[b]
        row = ml_ref[pl.ds(tid, 1), :]
        p = jax.nn.sigmoid(row)
        w_ref[pl.ds(b, 1), :] = p / (jnp.sum(p) + EPS)


def _fused_body(wts_ref, x_ref, w_ref, bias_ref, la_ref, lb_ref, out_ref,
                weff_ref):
    b = pl.program_id(0)
    s = pl.program_id(1)

    @pl.when(s == 0)
    def _build_weff():
        A = la_ref[0] * wts_ref[b, 0]
        Bm = lb_ref[0] * wts_ref[b, 0]
        for k in range(1, N_SKILLS):
            A = A + la_ref[k] * wts_ref[b, k]
            Bm = Bm + lb_ref[k] * wts_ref[b, k]
        abT = jax.lax.dot_general(
            Bm, A, (((0,), (1,)), ((), ())),
            preferred_element_type=jnp.float32)
        weff_ref[...] = w_ref[...] + abT * (1.0 / RANK)

    out_ref[0] = jax.lax.dot_general(
        x_ref[0], weff_ref[...], (((1,), (1,)), ((), ())),
        preferred_element_type=jnp.float32) + bias_ref[...]


def kernel(x, task_ids, module_logits, weight, bias, lora_a, lora_b):
    bsz, seq, in_f = x.shape
    out_f = weight.shape[0]
    n_tasks, n_sk = module_logits.shape

    wts = pl.pallas_call(
        _router_body,
        in_specs=[
            pl.BlockSpec(memory_space=pltpu.SMEM),
            pl.BlockSpec(memory_space=pltpu.VMEM),
        ],
        out_specs=pl.BlockSpec(memory_space=pltpu.VMEM),
        out_shape=jax.ShapeDtypeStruct((bsz, n_sk), jnp.float32),
    )(task_ids.astype(jnp.int32), module_logits)

    bias2 = bias.reshape(1, out_f)
    la = lora_a.reshape(n_sk, in_f, RANK)
    lb = lora_b.reshape(n_sk, RANK, out_f)

    n_s = seq // TS
    out = pl.pallas_call(
        _fused_body,
        grid=(bsz, n_s),
        in_specs=[
            pl.BlockSpec(memory_space=pltpu.SMEM),
            pl.BlockSpec((1, TS, in_f), lambda b, s: (b, s, 0)),
            pl.BlockSpec((out_f, in_f), lambda b, s: (0, 0)),
            pl.BlockSpec((1, out_f), lambda b, s: (0, 0)),
            pl.BlockSpec((n_sk, in_f, RANK), lambda b, s: (0, 0, 0)),
            pl.BlockSpec((n_sk, RANK, out_f), lambda b, s: (0, 0, 0)),
        ],
        out_specs=pl.BlockSpec((1, TS, out_f), lambda b, s: (b, s, 0)),
        out_shape=jax.ShapeDtypeStruct((bsz, seq, out_f), jnp.float32),
        scratch_shapes=[pltpu.VMEM((out_f, in_f), jnp.float32)],
    )(wts, x, weight, bias2, la, lb)
    return out


# speedup vs baseline: 1.0576x; 1.0044x over previous
import jax
import jax.numpy as jnp
from jax.experimental import pallas as pl
from jax.experimental.pallas import tpu as pltpu

EPS = 1e-12
N_SKILLS = 8
RANK = 16
TS = 2048


def _router_body(task_ids_ref, ml_ref, w_ref):
    bsz = w_ref.shape[0]
    for b in range(bsz):
        tid = task_ids_ref[b]
        row = ml_ref[pl.ds(tid, 1), :]
        p = jax.nn.sigmoid(row)
        w_ref[pl.ds(b, 1), :] = p / (jnp.sum(p) + EPS)


def _fused_body(wts_ref, x_ref, w_ref, bias_ref, la_ref, lb_ref, out_ref,
                weff_ref):
    b = pl.program_id(0)
    s = pl.program_id(1)

    @pl.when(s == 0)
    def _build_weff():
        A = la_ref[0] * wts_ref[b, 0]
        Bm = lb_ref[0] * wts_ref[b, 0]
        for k in range(1, N_SKILLS):
            A = A + la_ref[k] * wts_ref[b, k]
            Bm = Bm + lb_ref[k] * wts_ref[b, k]
        abT = jax.lax.dot_general(
            Bm, A, (((0,), (1,)), ((), ())),
            preferred_element_type=jnp.float32)
        weff_ref[...] = w_ref[...] + abT * (1.0 / RANK)

    out_ref[0] = jax.lax.dot_general(
        x_ref[0], weff_ref[...], (((1,), (1,)), ((), ())),
        preferred_element_type=jnp.float32) + bias_ref[...]


def kernel(x, task_ids, module_logits, weight, bias, lora_a, lora_b):
    bsz, seq, in_f = x.shape
    out_f = weight.shape[0]
    n_tasks, n_sk = module_logits.shape

    wts = pl.pallas_call(
        _router_body,
        in_specs=[
            pl.BlockSpec(memory_space=pltpu.SMEM),
            pl.BlockSpec(memory_space=pltpu.VMEM),
        ],
        out_specs=pl.BlockSpec(memory_space=pltpu.VMEM),
        out_shape=jax.ShapeDtypeStruct((bsz, n_sk), jnp.float32),
    )(task_ids.astype(jnp.int32), module_logits)

    bias2 = bias.reshape(1, out_f)
    la = lora_a.reshape(n_sk, in_f, RANK)
    lb = lora_b.reshape(n_sk, RANK, out_f)

    n_s = seq // TS
    out = pl.pallas_call(
        _fused_body,
        grid=(bsz, n_s),
        in_specs=[
            pl.BlockSpec(memory_space=pltpu.SMEM),
            pl.BlockSpec((1, TS, in_f), lambda b, s: (b, s, 0)),
            pl.BlockSpec((out_f, in_f), lambda b, s: (0, 0)),
            pl.BlockSpec((1, out_f), lambda b, s: (0, 0)),
            pl.BlockSpec((n_sk, in_f, RANK), lambda b, s: (0, 0, 0)),
            pl.BlockSpec((n_sk, RANK, out_f), lambda b, s: (0, 0, 0)),
        ],
        out_specs=pl.BlockSpec((1, TS, out_f), lambda b, s: (b, s, 0)),
        out_shape=jax.ShapeDtypeStruct((bsz, seq, out_f), jnp.float32),
        scratch_shapes=[pltpu.VMEM((out_f, in_f), jnp.float32)],
        compiler_params=pltpu.CompilerParams(
            dimension_semantics=("parallel", "arbitrary")),
    )(wts, x, weight, bias2, la, lb)
    return out
